# baseline (speedup 1.0000x reference)
import functools

import jax
import jax.numpy as jnp
from jax import lax
from jax.experimental import pallas as pl
from jax.experimental.pallas import tpu as pltpu
from jax.experimental.pallas import tpu_sc as plsc

_NUM_CORES = 2
_NUM_SUBCORES = 16
_NW = _NUM_CORES * _NUM_SUBCORES

_CHUNK = 128


def _build_gather(n_idx: int, vocab: int, dim: int):
    assert n_idx % (_NW * _CHUNK) == 0
    chunks_per_w = n_idx // (_NW * _CHUNK)

    mesh = plsc.VectorSubcoreMesh(
        core_axis_name="c", subcore_axis_name="s",
        num_cores=_NUM_CORES, num_subcores=_NUM_SUBCORES)

    @functools.partial(
        pl.kernel,
        out_type=jax.ShapeDtypeStruct((n_idx, dim), jnp.float32),
        mesh=mesh,
        compiler_params=pltpu.CompilerParams(use_tc_tiling_on_sc=False),
        scratch_types=[
            pltpu.VMEM((chunks_per_w, _CHUNK), jnp.int32),
            pltpu.VMEM((_CHUNK, dim), jnp.float32),
            pltpu.SemaphoreType.DMA,
        ],
    )
    def gather(table_hbm, idx_hbm, out_hbm, idx_v, rows_v, sem):
        wid = lax.axis_index("s") * _NUM_CORES + lax.axis_index("c")
        cbase = wid * chunks_per_w
        pltpu.sync_copy(idx_hbm.at[wid], idx_v)

        @pl.loop(0, chunks_per_w)
        def _(c):
            pltpu.async_copy(table_hbm.at[idx_v.at[c]], rows_v, sem).wait()
            pltpu.sync_copy(
                rows_v, out_hbm.at[pl.ds((cbase + c) * _CHUNK, _CHUNK)])

    return gather


def kernel(x, table):
    b, l = x.shape
    vocab, dim = table.shape
    n_idx = b * l
    idx = x.reshape(_NW, n_idx // (_NW * _CHUNK), _CHUNK).astype(jnp.int32)
    rows = _build_gather(n_idx, vocab, dim)(table, idx)
    embeddings = rows.reshape(b, l, dim)
    mask = jnp.ones((b, l), dtype=x.dtype)
    return (embeddings, mask)


# speedup vs baseline: 1.9267x; 1.9267x over previous
import functools

import jax
import jax.numpy as jnp
from jax import lax
from jax.experimental import pallas as pl
from jax.experimental.pallas import tpu as pltpu
from jax.experimental.pallas import tpu_sc as plsc

_NUM_CORES = 2
_NUM_SUBCORES = 16
_NW = _NUM_CORES * _NUM_SUBCORES

_CHUNK = 128
_D0 = 256
_DT = 128


def _build_gather(n_idx: int, vocab: int, dim: int):
    assert n_idx % (_NW * _CHUNK) == 0
    n_chunks = n_idx // _CHUNK
    chunks_per_w = n_chunks // _NW
    dw = _D0 + _DT

    mesh = plsc.VectorSubcoreMesh(
        core_axis_name="c", subcore_axis_name="s",
        num_cores=_NUM_CORES, num_subcores=_NUM_SUBCORES)

    @functools.partial(
        pl.kernel,
        out_type=jax.ShapeDtypeStruct((n_chunks, _CHUNK, dw), jnp.float32),
        mesh=mesh,
        scratch_types=[
            pltpu.VMEM((chunks_per_w, _CHUNK), jnp.int32),
            pltpu.VMEM((_CHUNK, dw), jnp.float32),
            pltpu.SemaphoreType.DMA,
            pltpu.SemaphoreType.DMA,
        ],
    )
    def gather(table_hbm, tail_hbm, idx_hbm, out_hbm, idx_v, rows_v,
               sem_a, sem_b):
        wid = lax.axis_index("s") * _NUM_CORES + lax.axis_index("c")
        cbase = wid * chunks_per_w
        pltpu.sync_copy(idx_hbm.at[wid], idx_v)

        @pl.loop(0, chunks_per_w)
        def _(c):
            cp_a = pltpu.async_copy(
                table_hbm.at[idx_v.at[c], pl.ds(0, _D0)],
                rows_v.at[:, pl.ds(0, _D0)], sem_a)
            cp_b = pltpu.async_copy(
                tail_hbm.at[idx_v.at[c]],
                rows_v.at[:, pl.ds(_D0, _DT)], sem_b)
            cp_a.wait()
            cp_b.wait()
            pltpu.sync_copy(rows_v, out_hbm.at[cbase + c])

    return gather


def kernel(x, table):
    b, l = x.shape
    vocab, dim = table.shape
    n_idx = b * l
    idx = x.reshape(_NW, n_idx // (_NW * _CHUNK), _CHUNK).astype(jnp.int32)
    tail = jnp.pad(table[:, _D0:], ((0, 0), (0, _DT - (dim - _D0))))
    wide = _build_gather(n_idx, vocab, dim)(table, tail, idx)
    embeddings = wide.reshape(n_idx, _D0 + _DT)[:, :dim].reshape(b, l, dim)
    mask = jnp.ones((b, l), dtype=x.dtype)
    return (embeddings, mask)
